# baseline (device time: 60339 ns/iter reference)
import jax
import jax.numpy as jnp
from jax import lax
from jax.experimental import pallas as pl
from jax.experimental.pallas import tpu as pltpu

N_DEV = 32
LOG2 = 5
N_LAYERS = 3
K = 2


def _coords(i):
    z = i // 8
    t = i % 8
    y = t // 2
    x = (t % 2) ^ (y % 2)
    return x, y, z


def _logical(x, y, z):
    return z * 8 + 2 * y + (x ^ (y % 2))


_FLIPS = [(1, 0, 0), (0, 1, 0), (0, 2, 0), (0, 0, 1), (0, 0, 2)]

_ORDERS = ((0, 1, 2, 3, 4), (3, 0, 4, 1, 2))


def kernel(x, Win0, Wout0, Win1, Wout1, Win2, Wout2):
    b, d_shard = x.shape
    h = Win0.shape[1]
    hc = h // K

    def body(x_ref, win0_ref, wout0_ref, win1_ref, wout1_ref, win2_ref,
             wout2_ref, out_ref, acc_ref, send_ref, recv_ref, send_sems,
             recv_sems, ready_sems):
        my = lax.axis_index("i")
        mx, my_y, mz = _coords(my)
        partners = [
            _logical(mx ^ fx, my_y ^ fy, mz ^ fz) for fx, fy, fz in _FLIPS
        ]

        barrier_sem = pltpu.get_barrier_semaphore()
        pl.semaphore_signal(barrier_sem, inc=1)
        pl.semaphore_wait(barrier_sem, 1)
        for m, p in enumerate(partners):
            pl.semaphore_signal(
                ready_sems.at[m], inc=1,
                device_id=(p,), device_id_type=pl.DeviceIdType.MESH,
            )
        ready_waited = set()

        def ensure_ready(m):
            if m not in ready_waited:
                ready_waited.add(m)
                pl.semaphore_wait(ready_sems.at[m], 1)

        wins = [win0_ref, win1_ref, win2_ref]
        wouts = [wout0_ref, wout1_ref, wout2_ref]

        def slot(l, r, k):
            return (l * LOG2 + r) * K + k

        def exchange(l, j, k):
            ensure_ready(_ORDERS[k][j])
            return pltpu.make_async_remote_copy(
                src_ref=send_ref.at[k],
                dst_ref=recv_ref.at[slot(l, j, k)],
                send_sem=send_sems.at[slot(l, j, k)],
                recv_sem=recv_sems.at[slot(l, j, k)],
                device_id=(partners[_ORDERS[k][j]],),
                device_id_type=pl.DeviceIdType.MESH,
            )

        xv = x_ref[:, :]
        for l in range(N_LAYERS):
            rdmas = [None] * K
            for k in range(K):
                pk = jnp.dot(
                    xv, wins[l][:, k * hc:(k + 1) * hc],
                    preferred_element_type=jnp.float32,
                )
                acc_ref[k, :, :] = pk
                send_ref[k, :, :] = pk.astype(jnp.bfloat16)
                rd = exchange(l, 0, k)
                rd.start()
                rdmas[k] = rd
            for r in range(LOG2 - 1):
                for k in range(K):
                    rdmas[k].wait()
                    ak = acc_ref[k, :, :] + recv_ref[
                        slot(l, r, k), :, :
                    ].astype(jnp.float32)
                    acc_ref[k, :, :] = ak
                    send_ref[k, :, :] = ak.astype(jnp.bfloat16)
                    rd = exchange(l, r + 1, k)
                    rd.start()
                    rdmas[k] = rd
            r = LOG2 - 1
            xv = jnp.zeros((b, d_shard), jnp.float32)
            for k in range(K):
                rdmas[k].wait()
                hk = jnp.maximum(
                    acc_ref[k, :, :]
                    + recv_ref[slot(l, r, k), :, :].astype(jnp.float32),
                    0.0,
                )
                xv = xv + jnp.dot(
                    hk, wouts[l][k * hc:(k + 1) * hc, :],
                    preferred_element_type=jnp.float32,
                )
        out_ref[:, :] = xv

    n_slots = N_LAYERS * LOG2 * K
    return pl.pallas_call(
        body,
        out_shape=jax.ShapeDtypeStruct((b, d_shard), jnp.float32),
        in_specs=[pl.BlockSpec(memory_space=pltpu.VMEM)] * 7,
        out_specs=pl.BlockSpec(memory_space=pltpu.VMEM),
        scratch_shapes=[
            pltpu.VMEM((K, b, hc), jnp.float32),
            pltpu.VMEM((K, b, hc), jnp.bfloat16),
            pltpu.VMEM((n_slots, b, hc), jnp.bfloat16),
            pltpu.SemaphoreType.DMA((n_slots,)),
            pltpu.SemaphoreType.DMA((n_slots,)),
            pltpu.SemaphoreType.REGULAR((LOG2,)),
        ],
        compiler_params=pltpu.CompilerParams(collective_id=0),
    )(x, Win0, Wout0, Win1, Wout1, Win2, Wout2)


# device time: 59215 ns/iter; 1.0190x vs baseline; 1.0190x over previous
import jax
import jax.numpy as jnp
from jax import lax
from jax.experimental import pallas as pl
from jax.experimental.pallas import tpu as pltpu

N_DEV = 32
LOG2 = 5
N_LAYERS = 3
K = 2


def _coords(i):
    z = i // 8
    t = i % 8
    y = t // 2
    x = (t % 2) ^ (y % 2)
    return x, y, z


def _logical(x, y, z):
    return z * 8 + 2 * y + (x ^ (y % 2))


_FLIPS = [(1, 0, 0), (0, 1, 0), (0, 2, 0), (0, 0, 1), (0, 0, 2)]

_ORDERS = ((0, 1, 2, 3, 4), (3, 0, 4, 1, 2))


def kernel(x, Win0, Wout0, Win1, Wout1, Win2, Wout2):
    b, d_shard = x.shape
    h = Win0.shape[1]
    hc = h // K

    def body(x_ref, win0_ref, wout0_ref, win1_ref, wout1_ref, win2_ref,
             wout2_ref, out_ref, acc_ref, send_ref, recv_ref, send_sems,
             recv_sems):
        my = lax.axis_index("i")
        mx, my_y, mz = _coords(my)
        partners = [
            _logical(mx ^ fx, my_y ^ fy, mz ^ fz) for fx, fy, fz in _FLIPS
        ]

        barrier_sem = pltpu.get_barrier_semaphore()
        for p in partners:
            pl.semaphore_signal(
                barrier_sem, inc=1,
                device_id=(p,), device_id_type=pl.DeviceIdType.MESH,
            )
        pl.semaphore_wait(barrier_sem, LOG2)

        wins = [win0_ref, win1_ref, win2_ref]
        wouts = [wout0_ref, wout1_ref, wout2_ref]

        def slot(l, r, k):
            return (l * LOG2 + r) * K + k

        def exchange(l, j, k):
            return pltpu.make_async_remote_copy(
                src_ref=send_ref.at[k],
                dst_ref=recv_ref.at[slot(l, j, k)],
                send_sem=send_sems.at[slot(l, j, k)],
                recv_sem=recv_sems.at[slot(l, j, k)],
                device_id=(partners[_ORDERS[k][j]],),
                device_id_type=pl.DeviceIdType.MESH,
            )

        xv = x_ref[:, :]
        for l in range(N_LAYERS):
            rdmas = [None] * K
            for k in range(K):
                pk = jnp.dot(
                    xv, wins[l][:, k * hc:(k + 1) * hc],
                    preferred_element_type=jnp.float32,
                )
                acc_ref[k, :, :] = pk
                send_ref[k, :, :] = pk.astype(jnp.bfloat16)
                rd = exchange(l, 0, k)
                rd.start()
                rdmas[k] = rd
            for r in range(LOG2 - 1):
                for k in range(K):
                    rdmas[k].wait()
                    ak = acc_ref[k, :, :] + recv_ref[
                        slot(l, r, k), :, :
                    ].astype(jnp.float32)
                    acc_ref[k, :, :] = ak
                    send_ref[k, :, :] = ak.astype(jnp.bfloat16)
                    rd = exchange(l, r + 1, k)
                    rd.start()
                    rdmas[k] = rd
            r = LOG2 - 1
            xv = jnp.zeros((b, d_shard), jnp.float32)
            for k in range(K):
                rdmas[k].wait()
                hk = jnp.maximum(
                    acc_ref[k, :, :]
                    + recv_ref[slot(l, r, k), :, :].astype(jnp.float32),
                    0.0,
                )
                xv = xv + jnp.dot(
                    hk, wouts[l][k * hc:(k + 1) * hc, :],
                    preferred_element_type=jnp.float32,
                )
        out_ref[:, :] = xv

    n_slots = N_LAYERS * LOG2 * K
    return pl.pallas_call(
        body,
        out_shape=jax.ShapeDtypeStruct((b, d_shard), jnp.float32),
        in_specs=[pl.BlockSpec(memory_space=pltpu.VMEM)] * 7,
        out_specs=pl.BlockSpec(memory_space=pltpu.VMEM),
        scratch_shapes=[
            pltpu.VMEM((K, b, hc), jnp.float32),
            pltpu.VMEM((K, b, hc), jnp.bfloat16),
            pltpu.VMEM((n_slots, b, hc), jnp.bfloat16),
            pltpu.SemaphoreType.DMA((n_slots,)),
            pltpu.SemaphoreType.DMA((n_slots,)),
        ],
        compiler_params=pltpu.CompilerParams(collective_id=0),
    )(x, Win0, Wout0, Win1, Wout1, Win2, Wout2)
